# baseline (speedup 1.0000x reference)
import jax
import jax.numpy as jnp
from jax.experimental import pallas as pl


def _body(x_ref, p_ref, o_ref):
    o_ref[...] = x_ref[...] + p_ref[...]


def kernel(x, pos_table):
    B, S, D = x.shape
    S_TILE = 256
    pos = pos_table[:S]
    return pl.pallas_call(
        _body,
        grid=(S // S_TILE,),
        in_specs=[
            pl.BlockSpec((B, S_TILE, D), lambda i: (0, i, 0)),
            pl.BlockSpec((S_TILE, D), lambda i: (i, 0)),
        ],
        out_specs=pl.BlockSpec((B, S_TILE, D), lambda i: (0, i, 0)),
        out_shape=jax.ShapeDtypeStruct(x.shape, x.dtype),
    )(x, pos)
